# baseline (speedup 1.0000x reference)
import jax
import jax.numpy as jnp
from jax import lax
from jax.experimental import pallas as pl
from jax.experimental.pallas import tpu as pltpu
from jax.experimental.pallas import tpu_sc as plsc
import functools

B = 819200
D = 64
NC = 2
NS = 16
NW = NC * NS
B_PER_W = B // NW
NBG = 2
CHUNK = NBG * 128
N_CHUNKS = B_PER_W // CHUNK
G_TOT = B // 128
G_PER_W = B_PER_W // 128
BLK = 1024

_mesh = plsc.VectorSubcoreMesh(core_axis_name="c", subcore_axis_name="s")


@functools.partial(
    pl.kernel,
    mesh=_mesh,
    out_type=jax.ShapeDtypeStruct((8, G_TOT, 8, 128), jnp.float32),
    scratch_types=[
        pltpu.VMEM((B_PER_W,), jnp.int32),
        pltpu.VMEM((NBG, 128, D), jnp.float32),
        pltpu.VMEM((NBG, 128, D), jnp.float32),
        pltpu.VMEM((NBG, 8, 8, 128), jnp.float32),
        pltpu.VMEM((NBG, 8, 8, 128), jnp.float32),
        pltpu.HBM((NBG, 128, D), jnp.float32),
        pltpu.HBM((NBG, 8, 8, 128), jnp.float32),
        pltpu.SemaphoreType.DMA,
        pltpu.SemaphoreType.DMA,
        pltpu.SemaphoreType.DMA,
        pltpu.SemaphoreType.DMA,
    ],
    compiler_params=pltpu.CompilerParams(
        use_tc_tiling_on_sc=False, needs_layout_passes=False),
)
def _embed_gather_t(x_hbm, table_hbm, out4, idx_all, rows0, rows1,
                    wt0, wt1, dummy_r, dummy_w, g0, g1, w0, w1):
    wid = lax.axis_index("s") * NC + lax.axis_index("c")
    base = wid * B_PER_W
    pltpu.sync_copy(x_hbm.at[pl.ds(base, B_PER_W)], idx_all)

    rows = (rows0, rows1)
    wt = (wt0, wt1)
    gsem = (g0, g1)
    wsem = (w0, w1)

    iota = lax.iota(jnp.int32, 16)
    d_hi = [(16 * k + iota) >> 3 for k in range(4)]
    d_lo = [(16 * k + iota) & 7 for k in range(4)]

    def gstart(j, p):
        for g in range(NBG):
            pltpu.async_copy(
                table_hbm.at[idx_all.at[pl.ds(j * CHUNK + g * 128, 128)]],
                rows[p].at[g], gsem[p])

    gstart(0, 0)

    @pl.loop(0, N_CHUNKS, step=2)
    def _pair(i):
        for b in range(2):
            j = i + b
            p = b
            q = 1 - b

            @pl.when(j + 1 < N_CHUNKS)
            def _prefetch():
                gstart(j + 1, q)

            pltpu.make_async_copy(dummy_r, rows[p], gsem[p]).wait()
            @pl.when(j >= 2)
            def _drain_wb():
                pltpu.make_async_copy(dummy_w, wt[p], wsem[p]).wait()

            for g in range(NBG):
                @plsc.parallel_loop(0, 128, unroll=8)
                def _tok(bp):
                    tok = lax.broadcast(bp, (16,))
                    for k in range(4):
                        v = rows[p][g, bp, pl.ds(16 * k, 16)]
                        plsc.store_scatter(
                            wt[p].at[g], [d_hi[k], d_lo[k], tok], v)

            bgo = wid * G_PER_W + j * NBG
            for g in range(NBG):
                pltpu.async_copy(
                    wt[p].at[g], out4.at[:, bgo + g, :, :], wsem[p])

    pltpu.make_async_copy(dummy_w, wt[0], wsem[0]).wait()
    pltpu.make_async_copy(dummy_w, wt[1], wsem[1]).wait()


def kernel(x, table):
    out4 = _embed_gather_t(x.astype(jnp.int32), table)
    return out4.transpose(1, 3, 0, 2).reshape(B, D)


# speedup vs baseline: 3.2137x; 3.2137x over previous
import jax
import jax.numpy as jnp
from jax import lax
from jax.experimental import pallas as pl
from jax.experimental.pallas import tpu as pltpu
from jax.experimental.pallas import tpu_sc as plsc
import functools

B = 819200
D = 64
NC = 2
NS = 16
NW = NC * NS
B_PER_W = B // NW
NBG = 2
CHUNK = NBG * 128
N_CHUNKS = B_PER_W // CHUNK
G_TOT = B // 128
G_PER_W = B_PER_W // 128
BLK = 1024

_mesh = plsc.VectorSubcoreMesh(core_axis_name="c", subcore_axis_name="s")


@functools.partial(
    pl.kernel,
    mesh=_mesh,
    out_type=jax.ShapeDtypeStruct((8, G_TOT, 8, 128), jnp.float32),
    scratch_types=[
        pltpu.VMEM((B_PER_W,), jnp.int32),
        pltpu.VMEM((NBG, 128, D), jnp.float32),
        pltpu.VMEM((NBG, 128, D), jnp.float32),
        pltpu.VMEM((NBG, 128, 65), jnp.float32),
        pltpu.VMEM((NBG, 8, 8, 128), jnp.float32),
        pltpu.VMEM((NBG, 8, 8, 128), jnp.float32),
        pltpu.HBM((NBG, 128, D), jnp.float32),
        pltpu.HBM((NBG, 8, 8, 128), jnp.float32),
        pltpu.SemaphoreType.DMA,
        pltpu.SemaphoreType.DMA,
        pltpu.SemaphoreType.DMA,
        pltpu.SemaphoreType.DMA,
    ],
    compiler_params=pltpu.CompilerParams(
        use_tc_tiling_on_sc=False, needs_layout_passes=False),
)
def _embed_gather_t(x_hbm, table_hbm, out4, idx_all, rows0, rows1, r65,
                    wt0, wt1, dummy_r, dummy_w, g0, g1, w0, w1):
    wid = lax.axis_index("s") * NC + lax.axis_index("c")
    base = wid * B_PER_W
    pltpu.sync_copy(x_hbm.at[pl.ds(base, B_PER_W)], idx_all)

    rows = (rows0, rows1)
    wt = (wt0, wt1)
    gsem = (g0, g1)
    wsem = (w0, w1)

    iota = lax.iota(jnp.int32, 16)

    def gstart(j, p):
        for g in range(NBG):
            pltpu.async_copy(
                table_hbm.at[idx_all.at[pl.ds(j * CHUNK + g * 128, 128)]],
                rows[p].at[g], gsem[p])

    gstart(0, 0)

    @pl.loop(0, N_CHUNKS, step=2)
    def _pair(i):
        for b in range(2):
            j = i + b
            p = b
            q = 1 - b

            @pl.when(j + 1 < N_CHUNKS)
            def _prefetch():
                gstart(j + 1, q)

            pltpu.make_async_copy(dummy_r, rows[p], gsem[p]).wait()
            @pl.when(j >= 2)
            def _drain_wb():
                pltpu.make_async_copy(dummy_w, wt[p], wsem[p]).wait()

            for g in range(NBG):
                @plsc.parallel_loop(0, 128, unroll=8)
                def _cp(bp):
                    for k in range(4):
                        r65[g, bp, pl.ds(16 * k, 16)] = (
                            rows[p][g, bp, pl.ds(16 * k, 16)])
                @plsc.parallel_loop(0, D, unroll=2)
                def _d(d):
                    hi = d >> 3
                    lo = d & 7
                    dsp = lax.broadcast(d, (16,))
                    for kb in range(8):
                        v = plsc.load_gather(
                            r65.at[g], [kb * 16 + iota, dsp])
                        wt[p][g, hi, lo, pl.ds(kb * 16, 16)] = v

            bgo = wid * G_PER_W + j * NBG
            for g in range(NBG):
                pltpu.async_copy(
                    wt[p].at[g], out4.at[:, bgo + g, :, :], wsem[p])

    pltpu.make_async_copy(dummy_w, wt[0], wsem[0]).wait()
    pltpu.make_async_copy(dummy_w, wt[1], wsem[1]).wait()


def kernel(x, table):
    out4 = _embed_gather_t(x.astype(jnp.int32), table)
    return out4.transpose(1, 3, 0, 2).reshape(B, D)


# speedup vs baseline: 3.2409x; 1.0085x over previous
import jax
import jax.numpy as jnp
from jax import lax
from jax.experimental import pallas as pl
from jax.experimental.pallas import tpu as pltpu
from jax.experimental.pallas import tpu_sc as plsc
import functools

B = 819200
D = 64
NC = 2
NS = 16
NW = NC * NS
B_PER_W = B // NW
NBG = 2
CHUNK = NBG * 128
N_CHUNKS = B_PER_W // CHUNK
G_TOT = B // 128
G_PER_W = B_PER_W // 128
BLK = 1024

_mesh = plsc.VectorSubcoreMesh(core_axis_name="c", subcore_axis_name="s")


@functools.partial(
    pl.kernel,
    mesh=_mesh,
    out_type=jax.ShapeDtypeStruct((8, G_TOT, 8, 128), jnp.float32),
    scratch_types=[
        pltpu.VMEM((B_PER_W,), jnp.int32),
        pltpu.VMEM((NBG, 128, D), jnp.float32),
        pltpu.VMEM((NBG, 128, D), jnp.float32),
        pltpu.VMEM((NBG, 128, 65), jnp.float32),
        pltpu.VMEM((NBG, 8, 8, 128), jnp.float32),
        pltpu.VMEM((NBG, 8, 8, 128), jnp.float32),
        pltpu.HBM((NBG, 128, D), jnp.float32),
        pltpu.HBM((NBG, 8, 8, 128), jnp.float32),
        pltpu.SemaphoreType.DMA,
        pltpu.SemaphoreType.DMA,
        pltpu.SemaphoreType.DMA,
        pltpu.SemaphoreType.DMA,
    ],
    compiler_params=pltpu.CompilerParams(
        use_tc_tiling_on_sc=False, needs_layout_passes=False),
)
def _embed_gather_t(x_hbm, table_hbm, out4, idx_all, rows0, rows1, r65,
                    wt0, wt1, dummy_r, dummy_w, g0, g1, w0, w1):
    wid = lax.axis_index("s") * NC + lax.axis_index("c")
    base = wid * B_PER_W
    pltpu.sync_copy(x_hbm.at[pl.ds(base, B_PER_W)], idx_all)

    rows = (rows0, rows1)
    wt = (wt0, wt1)
    gsem = (g0, g1)
    wsem = (w0, w1)

    iota = lax.iota(jnp.int32, 16)

    def gstart(j, p):
        for g in range(NBG):
            pltpu.async_copy(
                table_hbm.at[idx_all.at[pl.ds(j * CHUNK + g * 128, 128)]],
                rows[p].at[g], gsem[p])

    gstart(0, 0)

    @pl.loop(0, N_CHUNKS, step=2)
    def _pair(i):
        for b in range(2):
            j = i + b
            p = b
            q = 1 - b

            @pl.when(j + 1 < N_CHUNKS)
            def _prefetch():
                gstart(j + 1, q)

            pltpu.make_async_copy(dummy_r, rows[p], gsem[p]).wait()
            @pl.when(j >= 2)
            def _drain_wb():
                pltpu.make_async_copy(dummy_w, wt[p], wsem[p]).wait()

            bgo = wid * G_PER_W + j * NBG
            for g in range(NBG):
                @plsc.parallel_loop(0, 128, unroll=8)
                def _cp(bp):
                    for k in range(4):
                        r65[g, bp, pl.ds(16 * k, 16)] = (
                            rows[p][g, bp, pl.ds(16 * k, 16)])
                @plsc.parallel_loop(0, D, unroll=4)
                def _d(d):
                    hi = d >> 3
                    lo = d & 7
                    dsp = lax.broadcast(d, (16,))
                    for kb in range(8):
                        v = plsc.load_gather(
                            r65.at[g], [kb * 16 + iota, dsp])
                        wt[p][g, hi, lo, pl.ds(kb * 16, 16)] = v
                pltpu.async_copy(
                    wt[p].at[g], out4.at[:, bgo + g, :, :], wsem[p])

    pltpu.make_async_copy(dummy_w, wt[0], wsem[0]).wait()
    pltpu.make_async_copy(dummy_w, wt[1], wsem[1]).wait()


def kernel(x, table):
    out4 = _embed_gather_t(x.astype(jnp.int32), table)
    return out4.transpose(1, 3, 0, 2).reshape(B, D)
